# baseline (speedup 1.0000x reference)
import jax
import jax.numpy as jnp
from jax.experimental import pallas as pl

B_, I_, F_, K_ = 2048, 8, 2048, 16
BB = 512


def _fused_kernel(x_ref, a_ref, b_ref, out_ref, inner_ref):
    a = a_ref[0]
    na = a * jax.lax.rsqrt(jnp.sum(a * a, axis=0, keepdims=True))
    inner = jnp.dot(x_ref[...], na, preferred_element_type=jnp.float32,
                    precision=jax.lax.Precision.HIGHEST)
    inner_ref[0] = inner
    out_ref[...] = jnp.dot(inner, b_ref[0], preferred_element_type=jnp.float32,
                           precision=jax.lax.Precision.HIGHEST)


def kernel(x, A, B):
    x2 = x.reshape(B_, I_ * F_)
    nb = B_ // BB
    out2, inner_im = pl.pallas_call(
        _fused_kernel,
        grid=(I_, nb),
        in_specs=[
            pl.BlockSpec((BB, F_), lambda i, b: (b, i)),
            pl.BlockSpec((1, F_, K_), lambda i, b: (i, 0, 0)),
            pl.BlockSpec((1, K_, F_), lambda i, b: (i, 0, 0)),
        ],
        out_specs=[
            pl.BlockSpec((BB, F_), lambda i, b: (b, i)),
            pl.BlockSpec((1, BB, K_), lambda i, b: (i, b, 0)),
        ],
        out_shape=[
            jax.ShapeDtypeStruct((B_, I_ * F_), jnp.float32),
            jax.ShapeDtypeStruct((I_, B_, K_), jnp.float32),
        ],
    )(x2, A, B)
    out = out2.reshape(B_, I_, F_)
    inner = inner_im.transpose(1, 0, 2)
    return (out, inner)


# speedup vs baseline: 2.0316x; 2.0316x over previous
import jax
import jax.numpy as jnp
from jax.experimental import pallas as pl
from jax.experimental.pallas import tpu as pltpu

B_, I_, F_, K_ = 2048, 8, 2048, 16
BB = 512


def _fused_kernel(x_ref, a_ref, b_ref, out_ref, inner_ref, na_ref):
    @pl.when(pl.program_id(1) == 0)
    def _():
        a = a_ref[0]
        na_ref[...] = a * jax.lax.rsqrt(jnp.sum(a * a, axis=0, keepdims=True))

    inner = jnp.dot(x_ref[...], na_ref[...],
                    preferred_element_type=jnp.float32)
    inner_ref[0] = inner
    out_ref[...] = jnp.dot(inner, b_ref[0],
                           preferred_element_type=jnp.float32)


def kernel(x, A, B):
    x2 = x.reshape(B_, I_ * F_)
    nb = B_ // BB
    out2, inner_im = pl.pallas_call(
        _fused_kernel,
        grid=(I_, nb),
        in_specs=[
            pl.BlockSpec((BB, F_), lambda i, b: (b, i)),
            pl.BlockSpec((1, F_, K_), lambda i, b: (i, 0, 0)),
            pl.BlockSpec((1, K_, F_), lambda i, b: (i, 0, 0)),
        ],
        out_specs=[
            pl.BlockSpec((BB, F_), lambda i, b: (b, i)),
            pl.BlockSpec((1, BB, K_), lambda i, b: (i, b, 0)),
        ],
        out_shape=[
            jax.ShapeDtypeStruct((B_, I_ * F_), jnp.float32),
            jax.ShapeDtypeStruct((I_, B_, K_), jnp.float32),
        ],
        scratch_shapes=[pltpu.VMEM((F_, K_), jnp.float32)],
    )(x2, A, B)
    out = out2.reshape(B_, I_, F_)
    inner = inner_im.transpose(1, 0, 2)
    return (out, inner)


# speedup vs baseline: 5.6482x; 2.7802x over previous
import jax
import jax.numpy as jnp
from jax.experimental import pallas as pl
from jax.experimental.pallas import tpu as pltpu

B_, I_, F_, K_ = 2048, 8, 2048, 16
RB = 1024


def _fused_kernel(x_ref, a_ref, b_ref, out_ref, inner_ref, w_ref):
    @pl.when(pl.program_id(0) == 0)
    def _():
        cols = []
        for i in range(I_):
            a = a_ref[i]
            cols.append(a * jax.lax.rsqrt(jnp.sum(a * a, axis=0,
                                                  keepdims=True)))
        w_ref[...] = jnp.concatenate(cols, axis=1)

    inner_full = jnp.dot(x_ref[...], w_ref[...],
                         preferred_element_type=jnp.float32)
    row_inst = jax.lax.broadcasted_iota(jnp.int32, (RB, I_ * K_), 0) % I_
    lane_inst = jax.lax.broadcasted_iota(jnp.int32, (RB, I_ * K_), 1) // K_
    inner_masked = jnp.where(row_inst == lane_inst, inner_full, 0.0)
    out_ref[...] = jnp.dot(inner_masked, b_ref[...],
                           preferred_element_type=jnp.float32)
    sel_row = jax.lax.broadcasted_iota(jnp.int32, (I_ * K_, K_), 0) % K_
    sel_col = jax.lax.broadcasted_iota(jnp.int32, (I_ * K_, K_), 1)
    sel = (sel_row == sel_col).astype(jnp.float32)
    inner_ref[...] = jnp.dot(inner_masked, sel,
                             preferred_element_type=jnp.float32)


def kernel(x, A, B):
    xf = x.reshape(B_ * I_, F_)
    bf = B.reshape(I_ * K_, F_)
    nr = (B_ * I_) // RB
    out2, inner2 = pl.pallas_call(
        _fused_kernel,
        grid=(nr,),
        in_specs=[
            pl.BlockSpec((RB, F_), lambda r: (r, 0)),
            pl.BlockSpec((I_, F_, K_), lambda r: (0, 0, 0)),
            pl.BlockSpec((I_ * K_, F_), lambda r: (0, 0)),
        ],
        out_specs=[
            pl.BlockSpec((RB, F_), lambda r: (r, 0)),
            pl.BlockSpec((RB, K_), lambda r: (r, 0)),
        ],
        out_shape=[
            jax.ShapeDtypeStruct((B_ * I_, F_), jnp.float32),
            jax.ShapeDtypeStruct((B_ * I_, K_), jnp.float32),
        ],
        scratch_shapes=[pltpu.VMEM((F_, I_ * K_), jnp.float32)],
    )(xf, A, bf)
    out = out2.reshape(B_, I_, F_)
    inner = inner2.reshape(B_, I_, K_)
    return (out, inner)
